# baseline (speedup 1.0000x reference)
import jax
import jax.numpy as jnp
from jax import lax
from jax.experimental import pallas as pl
from jax.experimental.pallas import tpu as pltpu
from jax.experimental.pallas import tpu_sc as plsc

_N = 16384
_TABLE = 1001
_NC = 2
_NS = 16
_NW = _NC * _NS
_L = 16
_PER_W = _N // _NW
_MAGIC = jnp.float32(1.5 * 2.0**23)


def _sc_body(t_hbm, betas_hbm, out_hbm, t_v, betas_v, out_v, sem_b, sem_t, sem_o):
    wid = lax.axis_index("s") * _NC + lax.axis_index("c")
    base = wid * _PER_W
    cb = pltpu.async_copy(betas_hbm, betas_v, sem_b)
    ct = pltpu.async_copy(t_hbm.at[pl.ds(base, _PER_W)], t_v, sem_t)
    ct.wait()
    idxs = []
    for i in range(_PER_W // _L):
        tv = t_v[pl.ds(i * _L, _L)]
        r = tv * jnp.float32(1000.0)
        r = (r + _MAGIC) - _MAGIC
        idxs.append(r.astype(jnp.int32))
    cb.wait()
    chunk = _PER_W // 4
    pending = []
    for h in range(4):
        for i in range(chunk // _L):
            j = h * (chunk // _L) + i
            out_v[pl.ds(j * _L, _L)] = plsc.load_gather(betas_v, [idxs[j]])
        pending.append(pltpu.async_copy(
            out_v.at[pl.ds(h * chunk, chunk)],
            out_hbm.at[pl.ds(base + h * chunk, chunk)],
            sem_o,
        ))
    for c in pending:
        c.wait()


def kernel(t_normalized, betas):
    t_flat = t_normalized.reshape(_N)
    mesh = plsc.VectorSubcoreMesh(core_axis_name="c", subcore_axis_name="s")
    out = pl.kernel(
        _sc_body,
        mesh=mesh,
        out_type=jax.ShapeDtypeStruct((_N,), jnp.float32),
        scratch_types=[
            pltpu.VMEM((_PER_W,), jnp.float32),
            pltpu.VMEM((_TABLE,), jnp.float32),
            pltpu.VMEM((_PER_W,), jnp.float32),
            pltpu.SemaphoreType.DMA,
            pltpu.SemaphoreType.DMA,
            pltpu.SemaphoreType.DMA,
        ],
        compiler_params=pltpu.CompilerParams(needs_layout_passes=False),
    )(t_flat, betas)
    return out.reshape(_N, 1)


# speedup vs baseline: 1.0237x; 1.0237x over previous
import jax
import jax.numpy as jnp
from jax import lax
from jax.experimental import pallas as pl
from jax.experimental.pallas import tpu as pltpu
from jax.experimental.pallas import tpu_sc as plsc

_N = 16384
_TABLE = 1001
_NC = 2
_NS = 16
_NW = _NC * _NS
_L = 16
_PER_W = _N // _NW
_MAGIC = jnp.float32(1.5 * 2.0**23)


def _sc_body(t_hbm, betas_hbm, out_hbm, t_v, betas_v, out_v, sem_b, sem_t, sem_o):
    wid = lax.axis_index("s") * _NC + lax.axis_index("c")
    base = wid * _PER_W
    cb = pltpu.async_copy(betas_hbm, betas_v, sem_b)
    ct = pltpu.async_copy(t_hbm.at[pl.ds(base, _PER_W)], t_v, sem_t)
    ct.wait()
    cb.wait()
    chunk = _PER_W // 4

    def _step(i, carry):
        off = i * _L
        tv = t_v[pl.ds(off, _L)]
        r = tv * jnp.float32(1000.0)
        r = (r + _MAGIC) - _MAGIC
        idx = r.astype(jnp.int32)
        out_v[pl.ds(off, _L)] = plsc.load_gather(betas_v, [idx])
        return carry

    pending = []
    for h in range(4):
        lax.fori_loop(h * (chunk // _L), (h + 1) * (chunk // _L), _step, 0,
                      unroll=4)
        pending.append(pltpu.async_copy(
            out_v.at[pl.ds(h * chunk, chunk)],
            out_hbm.at[pl.ds(base + h * chunk, chunk)],
            sem_o,
        ))
    for c in pending:
        c.wait()


def kernel(t_normalized, betas):
    t_flat = t_normalized.reshape(_N)
    mesh = plsc.VectorSubcoreMesh(core_axis_name="c", subcore_axis_name="s")
    out = pl.kernel(
        _sc_body,
        mesh=mesh,
        out_type=jax.ShapeDtypeStruct((_N,), jnp.float32),
        scratch_types=[
            pltpu.VMEM((_PER_W,), jnp.float32),
            pltpu.VMEM((_TABLE,), jnp.float32),
            pltpu.VMEM((_PER_W,), jnp.float32),
            pltpu.SemaphoreType.DMA,
            pltpu.SemaphoreType.DMA,
            pltpu.SemaphoreType.DMA,
        ],
        compiler_params=pltpu.CompilerParams(needs_layout_passes=False),
    )(t_flat, betas)
    return out.reshape(_N, 1)


# speedup vs baseline: 1.0851x; 1.0599x over previous
import jax
import jax.numpy as jnp
from jax import lax
from jax.experimental import pallas as pl
from jax.experimental.pallas import tpu as pltpu
from jax.experimental.pallas import tpu_sc as plsc

_N = 16384
_TABLE = 1001
_NC = 1
_NS = 16
_NW = _NC * _NS
_L = 16
_PER_W = _N // _NW
_MAGIC = jnp.float32(1.5 * 2.0**23)


def _sc_body(t_hbm, betas_hbm, out_hbm, t_v, betas_v, out_v, sem_b, sem_t, sem_o):
    wid = lax.axis_index("s") * _NC + lax.axis_index("c")
    base = wid * _PER_W
    cb = pltpu.async_copy(betas_hbm, betas_v, sem_b)
    ct = pltpu.async_copy(t_hbm.at[pl.ds(base, _PER_W)], t_v, sem_t)
    ct.wait()
    cb.wait()
    chunk = _PER_W // 4

    def _step(i, carry):
        off = i * _L
        tv = t_v[pl.ds(off, _L)]
        r = tv * jnp.float32(1000.0)
        r = (r + _MAGIC) - _MAGIC
        idx = r.astype(jnp.int32)
        out_v[pl.ds(off, _L)] = plsc.load_gather(betas_v, [idx])
        return carry

    pending = []
    for h in range(4):
        lax.fori_loop(h * (chunk // _L), (h + 1) * (chunk // _L), _step, 0,
                      unroll=4)
        pending.append(pltpu.async_copy(
            out_v.at[pl.ds(h * chunk, chunk)],
            out_hbm.at[pl.ds(base + h * chunk, chunk)],
            sem_o,
        ))
    for c in pending:
        c.wait()


def kernel(t_normalized, betas):
    t_flat = t_normalized.reshape(_N)
    mesh = plsc.VectorSubcoreMesh(core_axis_name="c", subcore_axis_name="s", num_cores=1)
    out = pl.kernel(
        _sc_body,
        mesh=mesh,
        out_type=jax.ShapeDtypeStruct((_N,), jnp.float32),
        scratch_types=[
            pltpu.VMEM((_PER_W,), jnp.float32),
            pltpu.VMEM((_TABLE,), jnp.float32),
            pltpu.VMEM((_PER_W,), jnp.float32),
            pltpu.SemaphoreType.DMA,
            pltpu.SemaphoreType.DMA,
            pltpu.SemaphoreType.DMA,
        ],
        compiler_params=pltpu.CompilerParams(needs_layout_passes=False),
    )(t_flat, betas)
    return out.reshape(_N, 1)
